# baseline (speedup 1.0000x reference)
import functools

import jax
import jax.numpy as jnp
from jax import lax
from jax.experimental import pallas as pl
from jax.experimental.pallas import tpu as pltpu
from jax.experimental.pallas import tpu_sc as plsc

D = 32
IB = 256
NBUF = 5
K = 2


@jax.jit
def _gather_sc(idx_t, weight):
    F, N = idx_t.shape
    info = plsc.get_sparse_core_info()
    NC, NS = info.num_cores, info.num_subcores
    NW = NC * NS
    blocks_per_field = N // IB
    n_items = F * blocks_per_field
    items_per_w = n_items // NW
    assert items_per_w % NBUF == 0 and NBUF > K

    mesh = plsc.VectorSubcoreMesh(core_axis_name="c", subcore_axis_name="s")

    @functools.partial(
        pl.kernel,
        mesh=mesh,
        compiler_params=pltpu.CompilerParams(use_tc_tiling_on_sc=False),
        out_type=jax.ShapeDtypeStruct((F * D, N, 1), jnp.float32),
        scratch_types=[
            pltpu.VMEM((NBUF, IB), jnp.int32),
            pltpu.VMEM((NBUF, IB, D), jnp.float32),
            pltpu.SemaphoreType.DMA((NBUF,)),
            pltpu.SemaphoreType.DMA((NBUF,)),
            pltpu.SemaphoreType.DMA((NBUF,)),
        ],
    )
    def k(idx_hbm, table_hbm, out_hbm, idx_v, rows_v, isem, gsem, ssem):
        wid = lax.axis_index("s") * NC + lax.axis_index("c")
        w0 = wid * items_per_w

        def item_pos(w):
            j = lax.div(w, blocks_per_field)
            i0 = pl.multiple_of(lax.rem(w, blocks_per_field) * IB, 8)
            return j, i0

        def start_fetch(w, b):
            j, i0 = item_pos(w)
            pltpu.async_copy(idx_hbm.at[j, pl.ds(i0, IB)], idx_v.at[b],
                             isem.at[b])

        def start_gather(b):
            pltpu.async_copy(
                table_hbm.at[idx_v.at[b]], rows_v.at[b], gsem.at[b]
            )

        def drain_fetch(b):
            pltpu.make_async_copy(idx_hbm.at[0, pl.ds(0, IB)], idx_v.at[b],
                                  isem.at[b]).wait()

        def drain_gather(b):
            pltpu.make_async_copy(
                table_hbm.at[idx_v.at[b]], rows_v.at[b], gsem.at[b]
            ).wait()

        def start_scatter(w, b):
            j, i0 = item_pos(w)
            for f in range(D):
                pltpu.async_copy(
                    rows_v.at[b, pl.ds(0, IB), pl.ds(f, 1)],
                    out_hbm.at[j * D + f, pl.ds(i0, IB), pl.ds(0, 1)],
                    ssem.at[b],
                )

        def drain_scatter(b):
            for f in range(D):
                pltpu.make_async_copy(
                    idx_hbm.at[0, pl.ds(0, IB)],
                    idx_v.at[b],
                    ssem.at[b],
                ).wait()

        for b in range(K):
            start_fetch(w0 + b, b)
        for b in range(K):
            drain_fetch(b)
            start_gather(b)

        @pl.loop(0, items_per_w, step=NBUF)
        def _(t0):
            for b in range(NBUF):
                t = t0 + b
                tn = t + K
                bn = (b + K) % NBUF

                @pl.when(tn < items_per_w)
                def _():
                    @pl.when(tn >= NBUF)
                    def _():
                        drain_scatter(bn)

                    start_fetch(w0 + tn, bn)
                    drain_fetch(bn)
                    start_gather(bn)

                drain_gather(b)
                start_scatter(w0 + t, b)

        for b in range(NBUF):
            drain_scatter(b)

    return k(idx_t, weight)


def kernel(input, weight):
    NB, F = input.shape
    out2 = _gather_sc(input.T, weight)
    return jnp.transpose(out2.reshape(F, D, NB), (2, 0, 1))


# speedup vs baseline: 38.1668x; 38.1668x over previous
import functools

import jax
import jax.numpy as jnp
from jax import lax
from jax.experimental import pallas as pl
from jax.experimental.pallas import tpu as pltpu
from jax.experimental.pallas import tpu_sc as plsc

D = 32
NBUF = 4
K = 2


@jax.jit
def _gather_sc(idx, weight):
    NB, F = idx.shape
    info = plsc.get_sparse_core_info()
    NC, NS = info.num_cores, info.num_subcores
    NW = NC * NS
    rows_per_w = NB // NW
    assert rows_per_w % NBUF == 0 and NBUF > K

    mesh = plsc.VectorSubcoreMesh(core_axis_name="c", subcore_axis_name="s")

    @functools.partial(
        pl.kernel,
        mesh=mesh,
        compiler_params=pltpu.CompilerParams(use_tc_tiling_on_sc=False),
        out_type=jax.ShapeDtypeStruct((NB, F, D), jnp.float32),
        scratch_types=[
            pltpu.VMEM((NBUF, F), jnp.int32),
            pltpu.VMEM((NBUF, F, D), jnp.float32),
            pltpu.SemaphoreType.DMA((NBUF,)),
            pltpu.SemaphoreType.DMA((NBUF,)),
            pltpu.SemaphoreType.DMA((NBUF,)),
        ],
    )
    def k(idx_hbm, table_hbm, out_hbm, idx_v, rows_v, isem, gsem, ssem):
        wid = lax.axis_index("s") * NC + lax.axis_index("c")
        i0 = wid * rows_per_w

        def start_fetch(t, b):
            pltpu.async_copy(idx_hbm.at[i0 + t], idx_v.at[b], isem.at[b])

        def drain_fetch(b):
            pltpu.make_async_copy(idx_hbm.at[0], idx_v.at[b],
                                  isem.at[b]).wait()

        def start_gather(b):
            pltpu.async_copy(
                table_hbm.at[idx_v.at[b]], rows_v.at[b], gsem.at[b]
            )

        def drain_gather(b):
            pltpu.make_async_copy(
                table_hbm.at[idx_v.at[b]], rows_v.at[b], gsem.at[b]
            ).wait()

        def start_scatter(t, b):
            pltpu.async_copy(rows_v.at[b], out_hbm.at[i0 + t], ssem.at[b])

        def drain_scatter(b):
            pltpu.make_async_copy(rows_v.at[b], out_hbm.at[0],
                                  ssem.at[b]).wait()

        for b in range(K):
            start_fetch(b, b)
        for b in range(K):
            drain_fetch(b)
            start_gather(b)

        @pl.loop(0, rows_per_w, step=NBUF)
        def _(t0):
            for b in range(NBUF):
                t = t0 + b
                tn = t + K
                bn = (b + K) % NBUF

                @pl.when(tn < rows_per_w)
                def _():
                    @pl.when(tn >= NBUF)
                    def _():
                        drain_scatter(bn)

                    start_fetch(tn, bn)
                    drain_fetch(bn)
                    start_gather(bn)

                drain_gather(b)
                start_scatter(t, b)

        for b in range(NBUF):
            drain_scatter(b)

    return k(idx, weight)


def kernel(input, weight):
    return _gather_sc(input, weight)


# speedup vs baseline: 40.1780x; 1.0527x over previous
import functools

import jax
import jax.numpy as jnp
from jax import lax
from jax.experimental import pallas as pl
from jax.experimental.pallas import tpu as pltpu
from jax.experimental.pallas import tpu_sc as plsc

D = 32
RPI = 4
NBUF = 4
K = 2


@jax.jit
def _gather_sc(idx, weight):
    NB, F = idx.shape
    info = plsc.get_sparse_core_info()
    NC, NS = info.num_cores, info.num_subcores
    NW = NC * NS
    rows_per_w = NB // NW
    items_per_w = rows_per_w // RPI
    assert items_per_w % NBUF == 0 and NBUF > K

    mesh = plsc.VectorSubcoreMesh(core_axis_name="c", subcore_axis_name="s")

    @functools.partial(
        pl.kernel,
        mesh=mesh,
        compiler_params=pltpu.CompilerParams(use_tc_tiling_on_sc=False),
        out_type=jax.ShapeDtypeStruct((NB, F, D), jnp.float32),
        scratch_types=[
            pltpu.VMEM((rows_per_w, F), jnp.int32),
            pltpu.VMEM((NBUF, RPI, F, D), jnp.float32),
            pltpu.SemaphoreType.DMA((NBUF,)),
            pltpu.SemaphoreType.DMA((NBUF,)),
        ],
    )
    def k(idx_hbm, table_hbm, out_hbm, idx_v, rows_v, gsem, ssem):
        wid = lax.axis_index("s") * NC + lax.axis_index("c")
        i0 = wid * rows_per_w

        pltpu.sync_copy(idx_hbm.at[pl.ds(i0, rows_per_w)], idx_v)

        def start_gathers(t, b):
            for q in range(RPI):
                pltpu.async_copy(
                    table_hbm.at[idx_v.at[t * RPI + q]],
                    rows_v.at[b, q],
                    gsem.at[b],
                )

        def drain_gathers(t, b):
            for q in range(RPI):
                pltpu.make_async_copy(
                    table_hbm.at[idx_v.at[t * RPI + q]],
                    rows_v.at[b, q],
                    gsem.at[b],
                ).wait()

        def start_scatter(t, b):
            pltpu.async_copy(
                rows_v.at[b],
                out_hbm.at[pl.ds(i0 + t * RPI, RPI)],
                ssem.at[b],
            )

        def drain_scatter(b):
            pltpu.make_async_copy(
                rows_v.at[b],
                out_hbm.at[pl.ds(0, RPI)],
                ssem.at[b],
            ).wait()

        for b in range(K):
            start_gathers(b, b)

        @pl.loop(0, items_per_w, step=NBUF)
        def _(t0):
            for b in range(NBUF):
                t = t0 + b
                tn = t + K
                bn = (b + K) % NBUF

                @pl.when(tn < items_per_w)
                def _():
                    @pl.when(tn >= NBUF)
                    def _():
                        drain_scatter(bn)

                    start_gathers(tn, bn)

                drain_gathers(t, b)
                start_scatter(t, b)

        for b in range(NBUF):
            drain_scatter(b)

    return k(idx, weight)


def kernel(input, weight):
    return _gather_sc(input, weight)
